# baseline (speedup 1.0000x reference)
import functools
import math

import jax
import jax.numpy as jnp
import numpy as np
from jax import lax
from jax.experimental import pallas as pl
from jax.experimental.pallas import tpu as pltpu
from jax.experimental.pallas import tpu_sc as plsc

B, L = 1024, 200
EMBED = 128
HOUR_EMBED = EMBED // 4
N = B * L

NSPLIT = 4
BH = B // NSPLIT
NH = N // NSPLIT

NC, NS = 2, 16
NW = NC * NS
NPW = NH // NW
CHUNK = 64
NCHUNK = NPW // CHUNK
NBUF = 4

_HI = np.int32(-65536)


def _make_sc_gather():
    mesh = plsc.VectorSubcoreMesh(core_axis_name="c", subcore_axis_name="s")

    @functools.partial(
        pl.kernel,
        mesh=mesh,
        out_type=jax.ShapeDtypeStruct((NH // 2, EMBED), jnp.int32),
        scratch_types=[
            pltpu.VMEM((NCHUNK, CHUNK), jnp.int32),
        ]
        + [pltpu.VMEM((CHUNK, EMBED), jnp.float32) for _ in range(NBUF)]
        + [pltpu.VMEM((CHUNK // 2, EMBED), jnp.int32) for _ in range(NBUF)]
        + [pltpu.SemaphoreType.DMA for _ in range(2 * NBUF)],
    )
    def sc_gather(table_hbm, idx_hbm, out_hbm, idx_v, *bufs):
        rows = bufs[:NBUF]
        prows = bufs[NBUF : 2 * NBUF]
        gsem = bufs[2 * NBUF : 3 * NBUF]
        osem = bufs[3 * NBUF : 4 * NBUF]
        wid = lax.axis_index("s") * NC + lax.axis_index("c")
        base = wid * (NPW // 2)
        pltpu.sync_copy(idx_hbm.at[wid], idx_v)

        def out_at(j):
            return out_hbm.at[pl.ds(base + j * (CHUNK // 2), CHUNK // 2)]

        def convert(b):
            @plsc.parallel_loop(0, CHUNK // 16, unroll=2)
            def cbody(t):
                r0 = 16 * t
                w0 = 8 * t
                for s in range(8):
                    for g in range(EMBED // 16):
                        a = rows[b][r0 + s, pl.ds(16 * g, 16)]
                        c = rows[b][r0 + s + 8, pl.ds(16 * g, 16)]
                        au = lax.bitcast_convert_type(a, jnp.int32)
                        cu = lax.bitcast_convert_type(c, jnp.int32)
                        prows[b][w0 + s, pl.ds(16 * g, 16)] = (
                            (cu & _HI) | lax.shift_right_logical(au, 16)
                        )

        for b in range(NBUF - 1):
            pltpu.async_copy(table_hbm.at[idx_v.at[b]], rows[b], gsem[b])

        def step(j, b):
            nb = (b + NBUF - 1) % NBUF

            @pl.when(j + NBUF - 1 < NCHUNK)
            def _():
                pltpu.async_copy(
                    table_hbm.at[idx_v.at[j + NBUF - 1]], rows[nb], gsem[nb]
                )

            pltpu.make_async_copy(
                table_hbm.at[idx_v.at[j]], rows[b], gsem[b]
            ).wait()

            @pl.when(j >= NBUF)
            def _():
                pltpu.make_async_copy(
                    prows[b], out_at(j - NBUF), osem[b]
                ).wait()

            convert(b)
            pltpu.async_copy(prows[b], out_at(j), osem[b])

        def body(jj, _):
            for b in range(NBUF):
                step(jj * NBUF + b, b)
            return 0

        lax.fori_loop(0, NCHUNK // NBUF, body, 0, unroll=False)
        for j in range(NCHUNK - NCHUNK % NBUF, NCHUNK):
            step(j, j % NBUF)
        for j in range(NCHUNK - NBUF, NCHUNK):
            b = j % NBUF
            pltpu.make_async_copy(prows[b], out_at(j), osem[b]).wait()

    return sc_gather


_sc_gather = _make_sc_gather()

BB = 32
NFB = BB * L


def _tc_body(tok_ref, hour_ref, ht_ref, w1_ref, w2_ref, peb_ref, out_ref):
    u = tok_ref[...]
    lo = lax.bitcast_convert_type(u << 16, jnp.float32)
    hi = lax.bitcast_convert_type(u & _HI, jnp.float32)
    w1 = w1_ref[...]
    mlo = jnp.dot(lo, w1, preferred_element_type=jnp.float32)
    mhi = jnp.dot(hi, w1, preferred_element_type=jnp.float32)
    tokacc = jnp.concatenate(
        [mlo.reshape(-1, 8, EMBED), mhi.reshape(-1, 8, EMBED)], axis=1
    ).reshape(NFB, EMBED)
    hour = hour_ref[...].reshape(NFB)
    onehot = (hour[:, None] == lax.broadcasted_iota(jnp.int32, (NFB, 32), 1))
    hcb = jnp.dot(ht_ref[...], w2_ref[...], preferred_element_type=jnp.float32)
    acc = tokacc + jnp.dot(
        onehot.astype(jnp.float32), hcb, preferred_element_type=jnp.float32
    )
    acc = acc + jnp.broadcast_to(
        peb_ref[...][None, :, :], (BB, L, EMBED)
    ).reshape(NFB, EMBED)
    out_ref[...] = jnp.tanh(acc)


def _tc_body2(dst_ref, *rest):
    del dst_ref
    _tc_body(*rest)


_TC_COMMON_SPECS = [
    pl.BlockSpec((1, 1, NFB), lambda i: (i, 0, 0)),
    pl.BlockSpec((32, HOUR_EMBED), lambda i: (0, 0)),
    pl.BlockSpec((EMBED, EMBED), lambda i: (0, 0)),
    pl.BlockSpec((HOUR_EMBED, EMBED), lambda i: (0, 0)),
    pl.BlockSpec((L, EMBED), lambda i: (0, 0)),
]


def _tc_fuse_first(tok, hour, ht, w1, w2, peb):
    return pl.pallas_call(
        _tc_body,
        grid=(NH // NFB,),
        in_specs=[pl.BlockSpec((NFB // 2, EMBED), lambda i: (i, 0))]
        + _TC_COMMON_SPECS,
        out_specs=pl.BlockSpec((NFB, EMBED), lambda i: (i, 0)),
        out_shape=jax.ShapeDtypeStruct((N, EMBED), jnp.float32),
    )(tok, hour, ht, w1, w2, peb)


def _tc_fuse_rest(dst, tok, hour, ht, w1, w2, peb, part):
    off = part * (NH // NFB)
    return pl.pallas_call(
        _tc_body2,
        grid=(NH // NFB,),
        in_specs=[
            pl.BlockSpec(memory_space=pl.ANY),
            pl.BlockSpec((NFB // 2, EMBED), lambda i: (i, 0)),
        ]
        + _TC_COMMON_SPECS,
        out_specs=pl.BlockSpec((NFB, EMBED), lambda i, off=off: (i + off, 0)),
        out_shape=jax.ShapeDtypeStruct((N, EMBED), jnp.float32),
        input_output_aliases={0: 0},
    )(dst, tok, hour, ht, w1, w2, peb)


def _sinusoidal_pe(seq_len, d_model):
    pos = jnp.arange(seq_len, dtype=jnp.float32)[:, None]
    div_term = jnp.exp(
        jnp.arange(0, d_model, 2, dtype=jnp.float32)
        * (-math.log(10000.0) / d_model)
    )
    pe = jnp.zeros((seq_len, d_model), dtype=jnp.float32)
    pe = pe.at[:, 0::2].set(jnp.sin(pos * div_term))
    pe = pe.at[:, 1::2].set(jnp.cos(pos * div_term))
    return pe


def kernel(token_seq, hour_seq, loc_table, hour_table, W, b):
    idx = token_seq.astype(jnp.int32).reshape(NSPLIT, NW, NCHUNK, CHUNK)
    hour = hour_seq.astype(jnp.int32).reshape(NSPLIT, NH // NFB, 1, NFB)
    peb = _sinusoidal_pe(L, EMBED) + b[None, :]
    w1 = W[:EMBED]
    w2 = W[EMBED:]

    toks = [_sc_gather(loc_table, idx[i]) for i in range(NSPLIT)]
    out = _tc_fuse_first(toks[0], hour[0], hour_table, w1, w2, peb)
    for i in range(1, NSPLIT):
        out = _tc_fuse_rest(out, toks[i], hour[i], hour_table, w1, w2, peb, i)
    return out.reshape(B, L, EMBED)


# speedup vs baseline: 1.2171x; 1.2171x over previous
import functools
import math

import jax
import jax.numpy as jnp
import numpy as np
from jax import lax
from jax.experimental import pallas as pl
from jax.experimental.pallas import tpu as pltpu
from jax.experimental.pallas import tpu_sc as plsc

B, L = 1024, 200
EMBED = 128
HOUR_EMBED = EMBED // 4
N = B * L

NSPLIT = 2
BH = B // NSPLIT
NH = N // NSPLIT

NC, NS = 2, 16
NW = NC * NS
NPW = NH // NW
CHUNK = 128
NCHUNK = NPW // CHUNK
NBUF = 4

_HI = np.int32(-65536)


def _make_sc_gather():
    mesh = plsc.VectorSubcoreMesh(core_axis_name="c", subcore_axis_name="s")

    @functools.partial(
        pl.kernel,
        mesh=mesh,
        out_type=jax.ShapeDtypeStruct((NH // 2, EMBED), jnp.int32),
        scratch_types=[
            pltpu.VMEM((NCHUNK, CHUNK), jnp.int32),
        ]
        + [pltpu.VMEM((CHUNK, EMBED), jnp.float32) for _ in range(NBUF)]
        + [pltpu.VMEM((CHUNK // 2, EMBED), jnp.int32) for _ in range(NBUF)]
        + [pltpu.SemaphoreType.DMA for _ in range(2 * NBUF)],
    )
    def sc_gather(table_hbm, idx_hbm, out_hbm, idx_v, *bufs):
        rows = bufs[:NBUF]
        prows = bufs[NBUF : 2 * NBUF]
        gsem = bufs[2 * NBUF : 3 * NBUF]
        osem = bufs[3 * NBUF : 4 * NBUF]
        wid = lax.axis_index("s") * NC + lax.axis_index("c")
        base = wid * (NPW // 2)
        pltpu.sync_copy(idx_hbm.at[wid], idx_v)

        def out_at(j):
            return out_hbm.at[pl.ds(base + j * (CHUNK // 2), CHUNK // 2)]

        def convert(b):
            @plsc.parallel_loop(0, CHUNK // 16, unroll=2)
            def cbody(t):
                r0 = 16 * t
                w0 = 8 * t
                for s in range(8):
                    for g in range(EMBED // 16):
                        a = rows[b][r0 + s, pl.ds(16 * g, 16)]
                        c = rows[b][r0 + s + 8, pl.ds(16 * g, 16)]
                        au = lax.bitcast_convert_type(a, jnp.int32)
                        cu = lax.bitcast_convert_type(c, jnp.int32)
                        prows[b][w0 + s, pl.ds(16 * g, 16)] = (
                            (cu & _HI) | lax.shift_right_logical(au, 16)
                        )

        for b in range(NBUF - 1):
            pltpu.async_copy(table_hbm.at[idx_v.at[b]], rows[b], gsem[b])

        def step(j, b):
            nb = (b + NBUF - 1) % NBUF

            @pl.when(j + NBUF - 1 < NCHUNK)
            def _():
                pltpu.async_copy(
                    table_hbm.at[idx_v.at[j + NBUF - 1]], rows[nb], gsem[nb]
                )

            pltpu.make_async_copy(
                table_hbm.at[idx_v.at[j]], rows[b], gsem[b]
            ).wait()

            @pl.when(j >= NBUF)
            def _():
                pltpu.make_async_copy(
                    prows[b], out_at(j - NBUF), osem[b]
                ).wait()

            convert(b)
            pltpu.async_copy(prows[b], out_at(j), osem[b])

        def body(jj, _):
            for b in range(NBUF):
                step(jj * NBUF + b, b)
            return 0

        lax.fori_loop(0, NCHUNK // NBUF, body, 0, unroll=False)
        for j in range(NCHUNK - NCHUNK % NBUF, NCHUNK):
            step(j, j % NBUF)
        for j in range(NCHUNK - NBUF, NCHUNK):
            b = j % NBUF
            pltpu.make_async_copy(prows[b], out_at(j), osem[b]).wait()

    return sc_gather


_sc_gather = _make_sc_gather()

BB = 32
NFB = BB * L


def _tc_body(tok_ref, hour_ref, ht_ref, w1_ref, w2_ref, peb_ref, out_ref):
    u = tok_ref[...]
    lo = lax.bitcast_convert_type(u << 16, jnp.float32).astype(jnp.bfloat16)
    hi = lax.bitcast_convert_type(u & _HI, jnp.float32).astype(jnp.bfloat16)
    w1 = w1_ref[...].astype(jnp.bfloat16)
    mlo = jnp.dot(lo, w1, preferred_element_type=jnp.float32)
    mhi = jnp.dot(hi, w1, preferred_element_type=jnp.float32)
    tokacc = jnp.concatenate(
        [mlo.reshape(-1, 8, EMBED), mhi.reshape(-1, 8, EMBED)], axis=1
    ).reshape(NFB, EMBED)
    hour = hour_ref[...].reshape(NFB)
    onehot = (hour[:, None] == lax.broadcasted_iota(jnp.int32, (NFB, 32), 1))
    hcb = jnp.dot(ht_ref[...], w2_ref[...], preferred_element_type=jnp.float32)
    acc = tokacc + jnp.dot(
        onehot.astype(jnp.float32), hcb, preferred_element_type=jnp.float32
    )
    acc = acc + jnp.broadcast_to(
        peb_ref[...][None, :, :], (BB, L, EMBED)
    ).reshape(NFB, EMBED)
    out_ref[...] = jnp.tanh(acc)


def _tc_body2(dst_ref, *rest):
    del dst_ref
    _tc_body(*rest)


_TC_COMMON_SPECS = [
    pl.BlockSpec((1, 1, NFB), lambda i: (i, 0, 0)),
    pl.BlockSpec((32, HOUR_EMBED), lambda i: (0, 0)),
    pl.BlockSpec((EMBED, EMBED), lambda i: (0, 0)),
    pl.BlockSpec((HOUR_EMBED, EMBED), lambda i: (0, 0)),
    pl.BlockSpec((L, EMBED), lambda i: (0, 0)),
]


def _tc_fuse_first(tok, hour, ht, w1, w2, peb):
    return pl.pallas_call(
        _tc_body,
        grid=(NH // NFB,),
        in_specs=[pl.BlockSpec((NFB // 2, EMBED), lambda i: (i, 0))]
        + _TC_COMMON_SPECS,
        out_specs=pl.BlockSpec((NFB, EMBED), lambda i: (i, 0)),
        out_shape=jax.ShapeDtypeStruct((N, EMBED), jnp.float32),
    )(tok, hour, ht, w1, w2, peb)


def _tc_fuse_second(dst, tok, hour, ht, w1, w2, peb):
    off = NH // NFB
    return pl.pallas_call(
        _tc_body2,
        grid=(NH // NFB,),
        in_specs=[
            pl.BlockSpec(memory_space=pl.ANY),
            pl.BlockSpec((NFB // 2, EMBED), lambda i: (i, 0)),
        ]
        + _TC_COMMON_SPECS,
        out_specs=pl.BlockSpec((NFB, EMBED), lambda i: (i + off, 0)),
        out_shape=jax.ShapeDtypeStruct((N, EMBED), jnp.float32),
        input_output_aliases={0: 0},
    )(dst, tok, hour, ht, w1, w2, peb)


def _sinusoidal_pe(seq_len, d_model):
    pos = jnp.arange(seq_len, dtype=jnp.float32)[:, None]
    div_term = jnp.exp(
        jnp.arange(0, d_model, 2, dtype=jnp.float32)
        * (-math.log(10000.0) / d_model)
    )
    pe = jnp.zeros((seq_len, d_model), dtype=jnp.float32)
    pe = pe.at[:, 0::2].set(jnp.sin(pos * div_term))
    pe = pe.at[:, 1::2].set(jnp.cos(pos * div_term))
    return pe


def kernel(token_seq, hour_seq, loc_table, hour_table, W, b):
    idx = token_seq.astype(jnp.int32).reshape(NSPLIT, NW, NCHUNK, CHUNK)
    hour = hour_seq.astype(jnp.int32).reshape(NSPLIT, NH // NFB, 1, NFB)
    peb = _sinusoidal_pe(L, EMBED) + b[None, :]
    w1 = W[:EMBED]
    w2 = W[EMBED:]

    tok0 = _sc_gather(loc_table, idx[0])
    tok1 = _sc_gather(loc_table, idx[1])
    out = _tc_fuse_first(tok0, hour[0], hour_table, w1, w2, peb)
    out = _tc_fuse_second(out, tok1, hour[1], hour_table, w1, w2, peb)
    return out.reshape(B, L, EMBED)


# speedup vs baseline: 1.2580x; 1.0336x over previous
import functools
import math

import jax
import jax.numpy as jnp
import numpy as np
from jax import lax
from jax.experimental import pallas as pl
from jax.experimental.pallas import tpu as pltpu
from jax.experimental.pallas import tpu_sc as plsc

B, L = 1024, 200
EMBED = 128
HOUR_EMBED = EMBED // 4
N = B * L

NSPLIT = 2
BH = B // NSPLIT
NH = N // NSPLIT

NC, NS = 2, 16
NW = NC * NS
NPW = NH // NW
CHUNK = 128
NCHUNK = NPW // CHUNK
NBUF = 4

_HI = np.int32(-65536)


def _make_sc_gather():
    mesh = plsc.VectorSubcoreMesh(core_axis_name="c", subcore_axis_name="s")

    @functools.partial(
        pl.kernel,
        mesh=mesh,
        out_type=jax.ShapeDtypeStruct((NH // 2, EMBED), jnp.int32),
        scratch_types=[
            pltpu.VMEM((NCHUNK, CHUNK), jnp.int32),
        ]
        + [pltpu.VMEM((CHUNK, EMBED), jnp.float32) for _ in range(NBUF)]
        + [pltpu.VMEM((CHUNK // 2, EMBED), jnp.int32) for _ in range(NBUF)]
        + [pltpu.SemaphoreType.DMA for _ in range(2 * NBUF)],
    )
    def sc_gather(table_hbm, idx_hbm, out_hbm, idx_v, *bufs):
        rows = bufs[:NBUF]
        prows = bufs[NBUF : 2 * NBUF]
        gsem = bufs[2 * NBUF : 3 * NBUF]
        osem = bufs[3 * NBUF : 4 * NBUF]
        wid = lax.axis_index("s") * NC + lax.axis_index("c")
        base = wid * (NPW // 2)
        pltpu.sync_copy(idx_hbm.at[wid], idx_v)

        def out_at(j):
            return out_hbm.at[pl.ds(base + j * (CHUNK // 2), CHUNK // 2)]

        def convert(b):
            @plsc.parallel_loop(0, CHUNK // 16, unroll=2)
            def cbody(t):
                r0 = 16 * t
                w0 = 8 * t
                for s in range(8):
                    for g in range(EMBED // 16):
                        a = rows[b][r0 + s, pl.ds(16 * g, 16)]
                        c = rows[b][r0 + s + 8, pl.ds(16 * g, 16)]
                        au = lax.bitcast_convert_type(a, jnp.int32)
                        cu = lax.bitcast_convert_type(c, jnp.int32)
                        prows[b][w0 + s, pl.ds(16 * g, 16)] = (
                            (cu & _HI) | lax.shift_right_logical(au, 16)
                        )

        for b in range(NBUF - 1):
            pltpu.async_copy(table_hbm.at[idx_v.at[b]], rows[b], gsem[b])

        def step(j, b):
            nb = (b + NBUF - 1) % NBUF

            @pl.when(j + NBUF - 1 < NCHUNK)
            def _():
                pltpu.async_copy(
                    table_hbm.at[idx_v.at[j + NBUF - 1]], rows[nb], gsem[nb]
                )

            pltpu.make_async_copy(
                table_hbm.at[idx_v.at[j]], rows[b], gsem[b]
            ).wait()

            @pl.when(j >= NBUF)
            def _():
                pltpu.make_async_copy(
                    prows[b], out_at(j - NBUF), osem[b]
                ).wait()

            convert(b)
            pltpu.async_copy(prows[b], out_at(j), osem[b])

        def body(jj, _):
            for b in range(NBUF):
                step(jj * NBUF + b, b)
            return 0

        lax.fori_loop(0, NCHUNK // NBUF, body, 0, unroll=False)
        for j in range(NCHUNK - NCHUNK % NBUF, NCHUNK):
            step(j, j % NBUF)
        for j in range(NCHUNK - NBUF, NCHUNK):
            b = j % NBUF
            pltpu.make_async_copy(prows[b], out_at(j), osem[b]).wait()

    return sc_gather


_sc_gather = _make_sc_gather()

BB = 64
NFB = BB * L


def _tc_body(tok_ref, hour_ref, ht_ref, w1_ref, w2_ref, peb_ref, out_ref):
    u = tok_ref[...]
    lo = lax.bitcast_convert_type(u << 16, jnp.float32).astype(jnp.bfloat16)
    hi = lax.bitcast_convert_type(u & _HI, jnp.float32).astype(jnp.bfloat16)
    w1 = w1_ref[...].astype(jnp.bfloat16)
    mlo = jnp.dot(lo, w1, preferred_element_type=jnp.float32)
    mhi = jnp.dot(hi, w1, preferred_element_type=jnp.float32)
    tokacc = jnp.concatenate(
        [mlo.reshape(-1, 8, EMBED), mhi.reshape(-1, 8, EMBED)], axis=1
    ).reshape(NFB, EMBED)
    hour = hour_ref[...].reshape(NFB)
    onehot = (hour[:, None] == lax.broadcasted_iota(jnp.int32, (NFB, 32), 1))
    hcb = jnp.dot(ht_ref[...], w2_ref[...], preferred_element_type=jnp.float32)
    acc = tokacc + jnp.dot(
        onehot.astype(jnp.float32), hcb, preferred_element_type=jnp.float32
    )
    acc = acc + jnp.broadcast_to(
        peb_ref[...][None, :, :], (BB, L, EMBED)
    ).reshape(NFB, EMBED)
    out_ref[...] = jnp.tanh(acc)


def _tc_body2(dst_ref, *rest):
    del dst_ref
    _tc_body(*rest)


_TC_COMMON_SPECS = [
    pl.BlockSpec((1, 1, NFB), lambda i: (i, 0, 0)),
    pl.BlockSpec((32, HOUR_EMBED), lambda i: (0, 0)),
    pl.BlockSpec((EMBED, EMBED), lambda i: (0, 0)),
    pl.BlockSpec((HOUR_EMBED, EMBED), lambda i: (0, 0)),
    pl.BlockSpec((L, EMBED), lambda i: (0, 0)),
]


def _tc_fuse_first(tok, hour, ht, w1, w2, peb):
    return pl.pallas_call(
        _tc_body,
        grid=(NH // NFB,),
        in_specs=[pl.BlockSpec((NFB // 2, EMBED), lambda i: (i, 0))]
        + _TC_COMMON_SPECS,
        out_specs=pl.BlockSpec((NFB, EMBED), lambda i: (i, 0)),
        out_shape=jax.ShapeDtypeStruct((N, EMBED), jnp.float32),
    )(tok, hour, ht, w1, w2, peb)


def _tc_fuse_second(dst, tok, hour, ht, w1, w2, peb):
    off = NH // NFB
    return pl.pallas_call(
        _tc_body2,
        grid=(NH // NFB,),
        in_specs=[
            pl.BlockSpec(memory_space=pl.ANY),
            pl.BlockSpec((NFB // 2, EMBED), lambda i: (i, 0)),
        ]
        + _TC_COMMON_SPECS,
        out_specs=pl.BlockSpec((NFB, EMBED), lambda i: (i + off, 0)),
        out_shape=jax.ShapeDtypeStruct((N, EMBED), jnp.float32),
        input_output_aliases={0: 0},
    )(dst, tok, hour, ht, w1, w2, peb)


def _sinusoidal_pe(seq_len, d_model):
    pos = jnp.arange(seq_len, dtype=jnp.float32)[:, None]
    div_term = jnp.exp(
        jnp.arange(0, d_model, 2, dtype=jnp.float32)
        * (-math.log(10000.0) / d_model)
    )
    pe = jnp.zeros((seq_len, d_model), dtype=jnp.float32)
    pe = pe.at[:, 0::2].set(jnp.sin(pos * div_term))
    pe = pe.at[:, 1::2].set(jnp.cos(pos * div_term))
    return pe


def kernel(token_seq, hour_seq, loc_table, hour_table, W, b):
    idx = token_seq.astype(jnp.int32).reshape(NSPLIT, NW, NCHUNK, CHUNK)
    hour = hour_seq.astype(jnp.int32).reshape(NSPLIT, NH // NFB, 1, NFB)
    peb = _sinusoidal_pe(L, EMBED) + b[None, :]
    w1 = W[:EMBED]
    w2 = W[EMBED:]

    tok0 = _sc_gather(loc_table, idx[0])
    tok1 = _sc_gather(loc_table, idx[1])
    out = _tc_fuse_first(tok0, hour[0], hour_table, w1, w2, peb)
    out = _tc_fuse_second(out, tok1, hour[1], hour_table, w1, w2, peb)
    return out.reshape(B, L, EMBED)


# speedup vs baseline: 1.2786x; 1.0164x over previous
import functools
import math

import jax
import jax.numpy as jnp
import numpy as np
from jax import lax
from jax.experimental import pallas as pl
from jax.experimental.pallas import tpu as pltpu
from jax.experimental.pallas import tpu_sc as plsc

B, L = 1024, 200
EMBED = 128
HOUR_EMBED = EMBED // 4
N = B * L

NSPLIT = 2
BH = B // NSPLIT
NH = N // NSPLIT

NC, NS = 2, 16
NW = NC * NS
NPW = NH // NW
CHUNK = 128
NCHUNK = NPW // CHUNK
NBUF = 4

_HI = np.int32(-65536)


def _make_sc_gather():
    mesh = plsc.VectorSubcoreMesh(core_axis_name="c", subcore_axis_name="s")

    @functools.partial(
        pl.kernel,
        mesh=mesh,
        out_type=jax.ShapeDtypeStruct((NH // 2, EMBED), jnp.int32),
        scratch_types=[
            pltpu.VMEM((NCHUNK, CHUNK), jnp.int32),
        ]
        + [pltpu.VMEM((CHUNK, EMBED), jnp.float32) for _ in range(NBUF)]
        + [pltpu.VMEM((CHUNK // 2, EMBED), jnp.int32) for _ in range(NBUF)]
        + [pltpu.SemaphoreType.DMA for _ in range(2 * NBUF)],
    )
    def sc_gather(table_hbm, idx_hbm, out_hbm, idx_v, *bufs):
        rows = bufs[:NBUF]
        prows = bufs[NBUF : 2 * NBUF]
        gsem = bufs[2 * NBUF : 3 * NBUF]
        osem = bufs[3 * NBUF : 4 * NBUF]
        wid = lax.axis_index("s") * NC + lax.axis_index("c")
        base = wid * (NPW // 2)
        pltpu.sync_copy(idx_hbm.at[wid], idx_v)

        def out_at(j):
            return out_hbm.at[pl.ds(base + j * (CHUNK // 2), CHUNK // 2)]

        def convert(b):
            @plsc.parallel_loop(0, CHUNK // 16, unroll=2)
            def cbody(t):
                r0 = 16 * t
                w0 = 8 * t
                for s in range(8):
                    for g in range(EMBED // 16):
                        a = rows[b][r0 + s, pl.ds(16 * g, 16)]
                        c = rows[b][r0 + s + 8, pl.ds(16 * g, 16)]
                        au = lax.bitcast_convert_type(a, jnp.int32)
                        cu = lax.bitcast_convert_type(c, jnp.int32)
                        prows[b][w0 + s, pl.ds(16 * g, 16)] = (
                            (cu & _HI) | lax.shift_right_logical(au, 16)
                        )

        for b in range(NBUF - 1):
            pltpu.async_copy(table_hbm.at[idx_v.at[b]], rows[b], gsem[b])

        def step(j, b):
            nb = (b + NBUF - 1) % NBUF

            @pl.when(j + NBUF - 1 < NCHUNK)
            def _():
                pltpu.async_copy(
                    table_hbm.at[idx_v.at[j + NBUF - 1]], rows[nb], gsem[nb]
                )

            pltpu.make_async_copy(
                table_hbm.at[idx_v.at[j]], rows[b], gsem[b]
            ).wait()

            @pl.when(j >= NBUF)
            def _():
                pltpu.make_async_copy(
                    prows[b], out_at(j - NBUF), osem[b]
                ).wait()

            convert(b)
            pltpu.async_copy(prows[b], out_at(j), osem[b])

        def body(jj, _):
            for b in range(NBUF):
                step(jj * NBUF + b, b)
            return 0

        lax.fori_loop(0, NCHUNK // NBUF, body, 0, unroll=False)
        for j in range(NCHUNK - NCHUNK % NBUF, NCHUNK):
            step(j, j % NBUF)
        for j in range(NCHUNK - NBUF, NCHUNK):
            b = j % NBUF
            pltpu.make_async_copy(prows[b], out_at(j), osem[b]).wait()

    return sc_gather


_sc_gather = _make_sc_gather()

BB = 128
NFB = BB * L


def _tc_body(tok_ref, hour_ref, ht_ref, w1_ref, w2_ref, peb_ref, out_ref):
    u = tok_ref[...]
    lo = lax.bitcast_convert_type(u << 16, jnp.float32).astype(jnp.bfloat16)
    hi = lax.bitcast_convert_type(u & _HI, jnp.float32).astype(jnp.bfloat16)
    w1 = w1_ref[...].astype(jnp.bfloat16)
    mlo = jnp.dot(lo, w1, preferred_element_type=jnp.float32)
    mhi = jnp.dot(hi, w1, preferred_element_type=jnp.float32)
    tokacc = jnp.concatenate(
        [mlo.reshape(-1, 8, EMBED), mhi.reshape(-1, 8, EMBED)], axis=1
    ).reshape(NFB, EMBED)
    hour = hour_ref[...].reshape(NFB)
    onehot = (hour[:, None] == lax.broadcasted_iota(jnp.int32, (NFB, 32), 1))
    hcb = jnp.dot(ht_ref[...], w2_ref[...], preferred_element_type=jnp.float32)
    acc = tokacc + jnp.dot(
        onehot.astype(jnp.float32), hcb, preferred_element_type=jnp.float32
    )
    acc = acc + jnp.broadcast_to(
        peb_ref[...][None, :, :], (BB, L, EMBED)
    ).reshape(NFB, EMBED)
    out_ref[...] = jnp.tanh(acc)


def _tc_body2(dst_ref, *rest):
    del dst_ref
    _tc_body(*rest)


_TC_COMMON_SPECS = [
    pl.BlockSpec((1, 1, NFB), lambda i: (i, 0, 0)),
    pl.BlockSpec((32, HOUR_EMBED), lambda i: (0, 0)),
    pl.BlockSpec((EMBED, EMBED), lambda i: (0, 0)),
    pl.BlockSpec((HOUR_EMBED, EMBED), lambda i: (0, 0)),
    pl.BlockSpec((L, EMBED), lambda i: (0, 0)),
]


def _tc_fuse_first(tok, hour, ht, w1, w2, peb):
    return pl.pallas_call(
        _tc_body,
        grid=(NH // NFB,),
        in_specs=[pl.BlockSpec((NFB // 2, EMBED), lambda i: (i, 0))]
        + _TC_COMMON_SPECS,
        out_specs=pl.BlockSpec((NFB, EMBED), lambda i: (i, 0)),
        out_shape=jax.ShapeDtypeStruct((N, EMBED), jnp.float32),
    )(tok, hour, ht, w1, w2, peb)


def _tc_fuse_second(dst, tok, hour, ht, w1, w2, peb):
    off = NH // NFB
    return pl.pallas_call(
        _tc_body2,
        grid=(NH // NFB,),
        in_specs=[
            pl.BlockSpec(memory_space=pl.ANY),
            pl.BlockSpec((NFB // 2, EMBED), lambda i: (i, 0)),
        ]
        + _TC_COMMON_SPECS,
        out_specs=pl.BlockSpec((NFB, EMBED), lambda i: (i + off, 0)),
        out_shape=jax.ShapeDtypeStruct((N, EMBED), jnp.float32),
        input_output_aliases={0: 0},
    )(dst, tok, hour, ht, w1, w2, peb)


def _sinusoidal_pe(seq_len, d_model):
    pos = jnp.arange(seq_len, dtype=jnp.float32)[:, None]
    div_term = jnp.exp(
        jnp.arange(0, d_model, 2, dtype=jnp.float32)
        * (-math.log(10000.0) / d_model)
    )
    pe = jnp.zeros((seq_len, d_model), dtype=jnp.float32)
    pe = pe.at[:, 0::2].set(jnp.sin(pos * div_term))
    pe = pe.at[:, 1::2].set(jnp.cos(pos * div_term))
    return pe


def kernel(token_seq, hour_seq, loc_table, hour_table, W, b):
    idx = token_seq.astype(jnp.int32).reshape(NSPLIT, NW, NCHUNK, CHUNK)
    hour = hour_seq.astype(jnp.int32).reshape(NSPLIT, NH // NFB, 1, NFB)
    peb = _sinusoidal_pe(L, EMBED) + b[None, :]
    w1 = W[:EMBED]
    w2 = W[EMBED:]

    tok0 = _sc_gather(loc_table, idx[0])
    tok1 = _sc_gather(loc_table, idx[1])
    out = _tc_fuse_first(tok0, hour[0], hour_table, w1, w2, peb)
    out = _tc_fuse_second(out, tok1, hour[1], hour_table, w1, w2, peb)
    return out.reshape(B, L, EMBED)
